# Initial kernel scaffold; baseline (speedup 1.0000x reference)
import jax
import jax.numpy as jnp
from jax.experimental import pallas as pl


def kernel(x_user, x_issue, edge_src, edge_dst, W_mlp, b_mlp, Wrel_ui1, brel_ui1, Wroot_ui1, Wrel_iu1, brel_iu1, Wroot_iu1, Wrel_ui2, brel_ui2, Wroot_ui2, Wrel_iu2, brel_iu2, Wroot_iu2):
    raise NotImplementedError("write your pallas kernel here")


# speedup vs baseline: 1.6436x; 1.6436x over previous
import functools

import jax
import jax.numpy as jnp
from jax import lax
from jax.experimental import pallas as pl
from jax.experimental.pallas import tpu as pltpu
from jax.experimental.pallas import tpu_sc as plsc

NU = 50000
NI = 10000
E = 320000
D = 128

NC, NS = 2, 16
NW = NC * NS
CH = 128
EC = E // CH
CPT = -(-EC // NW)
EPAD = CPT * NW * CH
PAD = EPAD - E

RNG = 12544
NUP = 4 * RNG
ACCU_ROWS = RNG + 8
NIP = 10240
ACCI_ROWS = NIP


ZR = 64


def _zero_zbuf(zbuf):
    zv = jnp.zeros((16,), jnp.float32)

    def zrow(i, carry):
        for j in range(8):
            zbuf[i, pl.ds(j * 16, 16)] = zv
        return carry

    lax.fori_loop(0, ZR, zrow, 0)


def _seg10k_body(table, gidx, sidx, out, acc, gv, sv, rows, zbuf, dsem):
    c = lax.axis_index("c")
    s = lax.axis_index("s")
    _zero_zbuf(zbuf)
    base = s * 640
    for k in range(10):
        pltpu.sync_copy(zbuf, acc.at[pl.ds(base + k * ZR, ZR)])
    plsc.subcore_barrier()

    chunk0 = (c * NS + s) * CPT

    def step(i, carry):
        eb = (chunk0 + i) * CH
        pltpu.sync_copy(gidx.at[pl.ds(eb, CH)], gv)
        pltpu.sync_copy(sidx.at[pl.ds(eb, CH)], sv)
        pltpu.async_copy(table.at[gv], rows, dsem).wait()
        pltpu.sync_copy(rows, acc.at[sv], add=True)
        return carry

    lax.fori_loop(0, CPT, step, 0)
    plsc.subcore_barrier()
    pltpu.sync_copy(acc.at[pl.ds(base, 640)], out.at[c, pl.ds(base, 640)])


def _seg50k_body(table, gidx, sidx, out, acc, gv, sv, sv2, rows, zbuf, dsem):
    c = lax.axis_index("c")
    s = lax.axis_index("s")
    _zero_zbuf(zbuf)
    trash = RNG + (lax.iota(jnp.int32, 16) & 7)
    chunk0 = (c * NS + s) * CPT
    base = s * 784

    for r in range(4):
        for k in range(12):
            pltpu.sync_copy(zbuf, acc.at[pl.ds(base + k * ZR, ZR)])
        pltpu.sync_copy(zbuf.at[pl.ds(0, 16)], acc.at[pl.ds(base + 768, 16)])
        plsc.subcore_barrier()

        def step(i, carry):
            eb = (chunk0 + i) * CH
            pltpu.sync_copy(gidx.at[pl.ds(eb, CH)], gv)
            pltpu.sync_copy(sidx.at[pl.ds(eb, CH)], sv)
            pltpu.async_copy(table.at[gv], rows, dsem).wait()
            for j in range(8):
                v = sv[pl.ds(j * 16, 16)]
                loc = v - r * RNG
                ok = (loc >= 0) & (loc < RNG)
                sv2[pl.ds(j * 16, 16)] = jnp.where(ok, loc, trash)
            pltpu.sync_copy(rows, acc.at[sv2], add=True)
            return carry

        lax.fori_loop(0, CPT, step, 0)
        plsc.subcore_barrier()
        pltpu.sync_copy(acc.at[pl.ds(base, 784)],
                        out.at[c, pl.ds(r * RNG + base, 784)])
        plsc.subcore_barrier()


_MESH = plsc.VectorSubcoreMesh(core_axis_name="c", subcore_axis_name="s",
                               num_cores=NC, num_subcores=NS)


def _seg10k(table, gidx, sidx):
    return pl.kernel(
        _seg10k_body,
        out_type=jax.ShapeDtypeStruct((NC, NIP, D), jnp.float32),
        mesh=_MESH,
        scratch_types=[
            pltpu.VMEM_SHARED((ACCI_ROWS, D), jnp.float32),
            pltpu.VMEM((CH,), jnp.int32),
            pltpu.VMEM((CH,), jnp.int32),
            pltpu.VMEM((CH, D), jnp.float32),
            pltpu.VMEM((ZR, D), jnp.float32),
            pltpu.SemaphoreType.DMA,
        ],
    )(table, gidx, sidx)


def _seg50k(table, gidx, sidx):
    return pl.kernel(
        _seg50k_body,
        out_type=jax.ShapeDtypeStruct((NC, NUP, D), jnp.float32),
        mesh=_MESH,
        scratch_types=[
            pltpu.VMEM_SHARED((ACCU_ROWS, D), jnp.float32),
            pltpu.VMEM((CH,), jnp.int32),
            pltpu.VMEM((CH,), jnp.int32),
            pltpu.VMEM((CH,), jnp.int32),
            pltpu.VMEM((CH, D), jnp.float32),
            pltpu.VMEM((ZR, D), jnp.float32),
            pltpu.SemaphoreType.DMA,
        ],
    )(table, gidx, sidx)


def _lin_body(x_ref, w_ref, b_ref, o_ref):
    o_ref[...] = lax.dot_general(
        x_ref[...], w_ref[...], (((1,), (1,)), ((), ())),
        preferred_element_type=jnp.float32) + b_ref[...]


def _linear(x, w, b, bn=1000):
    n = x.shape[0]
    return pl.pallas_call(
        _lin_body,
        grid=(n // bn,),
        in_specs=[
            pl.BlockSpec((bn, D), lambda i: (i, 0)),
            pl.BlockSpec((D, D), lambda i: (0, 0)),
            pl.BlockSpec((1, D), lambda i: (0, 0)),
        ],
        out_specs=pl.BlockSpec((bn, D), lambda i: (i, 0)),
        out_shape=jax.ShapeDtypeStruct((n, D), jnp.float32),
    )(x, w, b.reshape(1, D))


def _fuse_body(p_ref, x_ref, wrel_ref, wroot_ref, b_ref, o_ref, *, relu):
    acc = p_ref[0] + p_ref[1]
    y = lax.dot_general(acc, wrel_ref[...], (((1,), (1,)), ((), ())),
                        preferred_element_type=jnp.float32)
    y = y + lax.dot_general(x_ref[...], wroot_ref[...], (((1,), (1,)), ((), ())),
                            preferred_element_type=jnp.float32)
    y = y + b_ref[...]
    if relu:
        y = jnp.maximum(y, 0.0)
    o_ref[...] = y


def _fuse(p, x, wrel, wroot, b, relu, bn=1000):
    n = x.shape[0]
    return pl.pallas_call(
        functools.partial(_fuse_body, relu=relu),
        grid=(n // bn,),
        in_specs=[
            pl.BlockSpec((NC, bn, D), lambda i: (0, i, 0)),
            pl.BlockSpec((bn, D), lambda i: (i, 0)),
            pl.BlockSpec((D, D), lambda i: (0, 0)),
            pl.BlockSpec((D, D), lambda i: (0, 0)),
            pl.BlockSpec((1, D), lambda i: (0, 0)),
        ],
        out_specs=pl.BlockSpec((bn, D), lambda i: (i, 0)),
        out_shape=jax.ShapeDtypeStruct((n, D), jnp.float32),
    )(p, x, wrel, wroot, b.reshape(1, D))


def kernel(x_user, x_issue, edge_src, edge_dst, W_mlp, b_mlp,
           Wrel_ui1, brel_ui1, Wroot_ui1, Wrel_iu1, brel_iu1, Wroot_iu1,
           Wrel_ui2, brel_ui2, Wroot_ui2, Wrel_iu2, brel_iu2, Wroot_iu2):
    es = edge_src.astype(jnp.int32)
    ed = edge_dst.astype(jnp.int32)
    ar = jnp.arange(PAD, dtype=jnp.int32)
    es_g = jnp.concatenate([es, (ar * 7919) % NU])
    ed_g = jnp.concatenate([ed, (ar * 127) % NI])
    es_s = jnp.concatenate([es, jnp.full((PAD,), NUP, jnp.int32)])
    ed_s = jnp.concatenate([ed, NI + (ar % 8)])

    xi = _linear(x_issue, W_mlp, b_mlp)
    Pi1 = _seg10k(x_user, es_g, ed_s)
    h_i = _fuse(Pi1, xi, Wrel_ui1, Wroot_ui1, brel_ui1, relu=True)
    Pu1 = _seg50k(xi, ed_g, es_s)
    h_u = _fuse(Pu1, x_user, Wrel_iu1, Wroot_iu1, brel_iu1, relu=True)
    Pi2 = _seg10k(h_u, es_g, ed_s)
    o_i = _fuse(Pi2, h_i, Wrel_ui2, Wroot_ui2, brel_ui2, relu=False)
    Pu2 = _seg50k(h_i, ed_g, es_s)
    o_u = _fuse(Pu2, h_u, Wrel_iu2, Wroot_iu2, brel_iu2, relu=False)
    return o_i, o_u


# speedup vs baseline: 1.7790x; 1.0824x over previous
import functools

import jax
import jax.numpy as jnp
from jax import lax
from jax.experimental import pallas as pl
from jax.experimental.pallas import tpu as pltpu
from jax.experimental.pallas import tpu_sc as plsc

NU = 50000
NI = 10000
E = 320000
D = 128

NC, NS = 2, 16
NW = NC * NS
CH = 128
EC = E // CH
CPT = -(-EC // NW)
EPAD = CPT * NW * CH
PAD = EPAD - E

RNG = 12544
NUP = 4 * RNG
ACCU_ROWS = RNG + 8
NIP = 10240
ACCI_ROWS = NIP


ZR = 64


def _zero_zbuf(zbuf):
    zv = jnp.zeros((16,), jnp.float32)

    def zrow(i, carry):
        for j in range(8):
            zbuf[i, pl.ds(j * 16, 16)] = zv
        return carry

    lax.fori_loop(0, ZR, zrow, 0)


def _pipe_chunks(n, table, idx_load, transform, scatter, gv, sv, rows, dsem):
    @pl.when(n > 0)
    def _prime():
        idx_load(0, 0)
        pltpu.async_copy(table.at[gv.at[0]], rows.at[0], dsem.at[0])

    def step(i, carry):
        j = lax.rem(i, 2)
        jn = lax.rem(i + 1, 2)

        @pl.when(i + 1 < n)
        def _prefetch():
            idx_load(i + 1, jn)
            pltpu.async_copy(table.at[gv.at[jn]], rows.at[jn], dsem.at[jn])

        pltpu.make_async_copy(table.at[gv.at[j]], rows.at[j], dsem.at[j]).wait()
        transform(j)
        scatter(j)
        return carry

    lax.fori_loop(0, n, step, 0)


def _seg10k_body(table, gidx, sidx, out, acc, gv, sv, rows, zbuf, dsem):
    c = lax.axis_index("c")
    s = lax.axis_index("s")
    _zero_zbuf(zbuf)
    base = s * 640
    for k in range(10):
        pltpu.sync_copy(zbuf, acc.at[pl.ds(base + k * ZR, ZR)])
    plsc.subcore_barrier()

    chunk0 = (c * NS + s) * CPT

    def idx_load(i, j):
        eb = (chunk0 + i) * CH
        pltpu.sync_copy(gidx.at[pl.ds(eb, CH)], gv.at[j])
        pltpu.sync_copy(sidx.at[pl.ds(eb, CH)], sv.at[j])

    def scatter(j):
        pltpu.sync_copy(rows.at[j], acc.at[sv.at[j]], add=True)

    _pipe_chunks(CPT, table, idx_load, lambda j: None, scatter,
                 gv, sv, rows, dsem)
    plsc.subcore_barrier()
    pltpu.sync_copy(acc.at[pl.ds(base, 640)], out.at[c, pl.ds(base, 640)])


def _seg50k_body(table, gidx, sidx, out, acc, gv, sv, sv2, rows, zbuf, dsem):
    c = lax.axis_index("c")
    s = lax.axis_index("s")
    _zero_zbuf(zbuf)
    trash = RNG + (lax.iota(jnp.int32, 16) & 7)
    chunk0 = (c * NS + s) * CPT
    base = s * 784

    for r in range(4):
        for k in range(12):
            pltpu.sync_copy(zbuf, acc.at[pl.ds(base + k * ZR, ZR)])
        pltpu.sync_copy(zbuf.at[pl.ds(0, 16)], acc.at[pl.ds(base + 768, 16)])
        plsc.subcore_barrier()

        def step(i, carry):
            eb = (chunk0 + i) * CH
            pltpu.sync_copy(gidx.at[pl.ds(eb, CH)], gv)
            pltpu.sync_copy(sidx.at[pl.ds(eb, CH)], sv)
            pltpu.async_copy(table.at[gv], rows, dsem).wait()
            for j in range(8):
                v = sv[pl.ds(j * 16, 16)]
                loc = v - r * RNG
                ok = (loc >= 0) & (loc < RNG)
                sv2[pl.ds(j * 16, 16)] = jnp.where(ok, loc, trash)
            pltpu.sync_copy(rows, acc.at[sv2], add=True)
            return carry

        lax.fori_loop(0, CPT, step, 0)
        plsc.subcore_barrier()
        pltpu.sync_copy(acc.at[pl.ds(base, 784)],
                        out.at[c, pl.ds(r * RNG + base, 784)])
        plsc.subcore_barrier()


_MESH = plsc.VectorSubcoreMesh(core_axis_name="c", subcore_axis_name="s",
                               num_cores=NC, num_subcores=NS)


def _seg10k(table, gidx, sidx):
    return pl.kernel(
        _seg10k_body,
        out_type=jax.ShapeDtypeStruct((NC, NIP, D), jnp.float32),
        mesh=_MESH,
        scratch_types=[
            pltpu.VMEM_SHARED((ACCI_ROWS, D), jnp.float32),
            pltpu.VMEM((2, CH), jnp.int32),
            pltpu.VMEM((2, CH), jnp.int32),
            pltpu.VMEM((2, CH, D), jnp.float32),
            pltpu.VMEM((ZR, D), jnp.float32),
            pltpu.SemaphoreType.DMA((2,)),
        ],
    )(table, gidx, sidx)


def _seg50k(table, gidx, sidx):
    return pl.kernel(
        _seg50k_body,
        out_type=jax.ShapeDtypeStruct((NC, NUP, D), jnp.float32),
        mesh=_MESH,
        scratch_types=[
            pltpu.VMEM_SHARED((ACCU_ROWS, D), jnp.float32),
            pltpu.VMEM((CH,), jnp.int32),
            pltpu.VMEM((CH,), jnp.int32),
            pltpu.VMEM((CH,), jnp.int32),
            pltpu.VMEM((CH, D), jnp.float32),
            pltpu.VMEM((ZR, D), jnp.float32),
            pltpu.SemaphoreType.DMA,
        ],
    )(table, gidx, sidx)


def _lin_body(x_ref, w_ref, b_ref, o_ref):
    o_ref[...] = lax.dot_general(
        x_ref[...], w_ref[...], (((1,), (1,)), ((), ())),
        preferred_element_type=jnp.float32) + b_ref[...]


def _linear(x, w, b, bn=1000):
    n = x.shape[0]
    return pl.pallas_call(
        _lin_body,
        grid=(n // bn,),
        in_specs=[
            pl.BlockSpec((bn, D), lambda i: (i, 0)),
            pl.BlockSpec((D, D), lambda i: (0, 0)),
            pl.BlockSpec((1, D), lambda i: (0, 0)),
        ],
        out_specs=pl.BlockSpec((bn, D), lambda i: (i, 0)),
        out_shape=jax.ShapeDtypeStruct((n, D), jnp.float32),
    )(x, w, b.reshape(1, D))


def _fuse_body(p_ref, x_ref, wrel_ref, wroot_ref, b_ref, o_ref, *, relu):
    acc = p_ref[0] + p_ref[1]
    y = lax.dot_general(acc, wrel_ref[...], (((1,), (1,)), ((), ())),
                        preferred_element_type=jnp.float32)
    y = y + lax.dot_general(x_ref[...], wroot_ref[...], (((1,), (1,)), ((), ())),
                            preferred_element_type=jnp.float32)
    y = y + b_ref[...]
    if relu:
        y = jnp.maximum(y, 0.0)
    o_ref[...] = y


def _fuse(p, x, wrel, wroot, b, relu, bn=1000):
    n = x.shape[0]
    return pl.pallas_call(
        functools.partial(_fuse_body, relu=relu),
        grid=(n // bn,),
        in_specs=[
            pl.BlockSpec((NC, bn, D), lambda i: (0, i, 0)),
            pl.BlockSpec((bn, D), lambda i: (i, 0)),
            pl.BlockSpec((D, D), lambda i: (0, 0)),
            pl.BlockSpec((D, D), lambda i: (0, 0)),
            pl.BlockSpec((1, D), lambda i: (0, 0)),
        ],
        out_specs=pl.BlockSpec((bn, D), lambda i: (i, 0)),
        out_shape=jax.ShapeDtypeStruct((n, D), jnp.float32),
    )(p, x, wrel, wroot, b.reshape(1, D))


def kernel(x_user, x_issue, edge_src, edge_dst, W_mlp, b_mlp,
           Wrel_ui1, brel_ui1, Wroot_ui1, Wrel_iu1, brel_iu1, Wroot_iu1,
           Wrel_ui2, brel_ui2, Wroot_ui2, Wrel_iu2, brel_iu2, Wroot_iu2):
    es = edge_src.astype(jnp.int32)
    ed = edge_dst.astype(jnp.int32)
    ar = jnp.arange(PAD, dtype=jnp.int32)
    es_g = jnp.concatenate([es, (ar * 7919) % NU])
    ed_g = jnp.concatenate([ed, (ar * 127) % NI])
    es_s = jnp.concatenate([es, jnp.full((PAD,), NUP, jnp.int32)])
    ed_s = jnp.concatenate([ed, NI + (ar % 8)])

    xi = _linear(x_issue, W_mlp, b_mlp)
    Pi1 = _seg10k(x_user, es_g, ed_s)
    h_i = _fuse(Pi1, xi, Wrel_ui1, Wroot_ui1, brel_ui1, relu=True)
    Pu1 = _seg50k(xi, ed_g, es_s)
    h_u = _fuse(Pu1, x_user, Wrel_iu1, Wroot_iu1, brel_iu1, relu=True)
    Pi2 = _seg10k(h_u, es_g, ed_s)
    o_i = _fuse(Pi2, h_i, Wrel_ui2, Wroot_ui2, brel_ui2, relu=False)
    Pu2 = _seg50k(h_i, ed_g, es_s)
    o_u = _fuse(Pu2, h_u, Wrel_iu2, Wroot_iu2, brel_iu2, relu=False)
    return o_i, o_u


# speedup vs baseline: 2.6015x; 1.4624x over previous
import functools

import jax
import jax.numpy as jnp
from jax import lax
from jax.experimental import pallas as pl
from jax.experimental.pallas import tpu as pltpu
from jax.experimental.pallas import tpu_sc as plsc

NU = 50000
NI = 10000
E = 320000
D = 128

NC, NS = 2, 16
NW = NC * NS
CH = 128
EC = E // CH
CPT = 2 * (-(-EC // (2 * NW)))
EPAD = CPT * NW * CH
PAD = EPAD - E

NIP = 10240
ACCI_ROWS = NIP

NR = 5
RNG = 10240
NUP = NR * RNG
SENT = NUP
TU = RNG // NS
ACCU_ROWS = RNG + 8

ZR = 64


def _zero_zbuf(zbuf):
    zv = jnp.zeros((16,), jnp.float32)

    def zrow(i, carry):
        for j in range(8):
            zbuf[i, pl.ds(j * 16, 16)] = zv
        return carry

    lax.fori_loop(0, ZR, zrow, 0)


def _seg10k_body(table, gidx, sidx, out, acc,
                 gva, gvb, sva, svb, rowsa, rowsb, zbuf,
                 dsa, dsb, ssa, ssb):
    c = lax.axis_index("c")
    s = lax.axis_index("s")
    _zero_zbuf(zbuf)
    base = s * 640
    for k in range(10):
        pltpu.sync_copy(zbuf, acc.at[pl.ds(base + k * ZR, ZR)])
    plsc.subcore_barrier()

    chunk0 = (c * NS + s) * CPT
    bufs = ((gva, sva, rowsa, dsa, ssa), (gvb, svb, rowsb, dsb, ssb))

    pltpu.sync_copy(gidx.at[pl.ds(chunk0 * CH, CH)], gva)
    pltpu.sync_copy(sidx.at[pl.ds(chunk0 * CH, CH)], sva)
    pltpu.async_copy(table.at[gva], rowsa, dsa)

    def half(i, jst):
        gv, sv, rows, ds_, ss = bufs[jst]
        gvn, svn, rowsn, dsn, ssn = bufs[1 - jst]

        @pl.when((i >= 1) & (i + 1 < CPT))
        def _wait_prev_scatter():
            pltpu.make_async_copy(rowsn, acc.at[svn], ssn).wait()

        @pl.when(i + 1 < CPT)
        def _prefetch():
            eb = (chunk0 + i + 1) * CH
            pltpu.sync_copy(gidx.at[pl.ds(eb, CH)], gvn)
            pltpu.sync_copy(sidx.at[pl.ds(eb, CH)], svn)
            pltpu.async_copy(table.at[gvn], rowsn, dsn)

        pltpu.make_async_copy(table.at[gv], rows, ds_).wait()
        pltpu.async_copy(rows, acc.at[sv], ss, add=True)

    def step2(i2, carry):
        half(i2 * 2, 0)
        half(i2 * 2 + 1, 1)
        return carry

    lax.fori_loop(0, CPT // 2, step2, 0)
    pltpu.make_async_copy(rowsa, acc.at[sva], ssa).wait()
    pltpu.make_async_copy(rowsb, acc.at[svb], ssb).wait()
    plsc.subcore_barrier()
    pltpu.sync_copy(acc.at[pl.ds(base, 640)], out.at[c, pl.ds(base, 640)])


def _seg50k_body(table, gidx, sidx, out, acc,
                 gva, gvb, sv, sv2a, sv2b, rowsa, rowsb, zbuf,
                 dsa, dsb, ssa, ssb):
    c = lax.axis_index("c")
    s = lax.axis_index("s")
    _zero_zbuf(zbuf)
    trash = RNG + (lax.iota(jnp.int32, 16) & 7)
    chunk0 = (c * NS + s) * CPT
    base = s * TU
    bufs = ((gva, sv2a, rowsa, dsa, ssa), (gvb, sv2b, rowsb, dsb, ssb))

    for r in range(NR):
        for k in range(TU // ZR):
            pltpu.sync_copy(zbuf, acc.at[pl.ds(base + k * ZR, ZR)])
        plsc.subcore_barrier()

        pltpu.sync_copy(gidx.at[pl.ds(chunk0 * CH, CH)], gva)
        pltpu.async_copy(table.at[gva], rowsa, dsa)

        def half(i, jst, r=r):
            gv, sv2, rows, ds_, ss = bufs[jst]
            gvn, sv2n, rowsn, dsn, ssn = bufs[1 - jst]

            @pl.when((i >= 1) & (i + 1 < CPT))
            def _wait_prev_scatter():
                pltpu.make_async_copy(rowsn, acc.at[sv2n], ssn).wait()

            @pl.when(i + 1 < CPT)
            def _prefetch():
                eb = (chunk0 + i + 1) * CH
                pltpu.sync_copy(gidx.at[pl.ds(eb, CH)], gvn)
                pltpu.async_copy(table.at[gvn], rowsn, dsn)

            eb = (chunk0 + i) * CH
            pltpu.sync_copy(sidx.at[pl.ds(eb, CH)], sv)
            for q in range(8):
                v = sv[pl.ds(q * 16, 16)]
                loc = v - r * RNG
                ok = (loc >= 0) & (loc < RNG)
                sv2[pl.ds(q * 16, 16)] = jnp.where(ok, loc, trash)
            pltpu.make_async_copy(table.at[gv], rows, ds_).wait()
            pltpu.async_copy(rows, acc.at[sv2], ss, add=True)

        def step2(i2, carry, r=r):
            half(i2 * 2, 0, r)
            half(i2 * 2 + 1, 1, r)
            return carry

        lax.fori_loop(0, CPT // 2, step2, 0)
        pltpu.make_async_copy(rowsa, acc.at[sv2a], ssa).wait()
        pltpu.make_async_copy(rowsb, acc.at[sv2b], ssb).wait()
        plsc.subcore_barrier()
        pltpu.sync_copy(acc.at[pl.ds(base, TU)],
                        out.at[c, pl.ds(r * RNG + base, TU)])
        plsc.subcore_barrier()


_MESH = plsc.VectorSubcoreMesh(core_axis_name="c", subcore_axis_name="s",
                               num_cores=NC, num_subcores=NS)


def _seg10k(table, gidx, sidx):
    return pl.kernel(
        _seg10k_body,
        out_type=jax.ShapeDtypeStruct((NC, NIP, D), jnp.float32),
        mesh=_MESH,
        scratch_types=[
            pltpu.VMEM_SHARED((ACCI_ROWS, D), jnp.float32),
            pltpu.VMEM((CH,), jnp.int32),
            pltpu.VMEM((CH,), jnp.int32),
            pltpu.VMEM((CH,), jnp.int32),
            pltpu.VMEM((CH,), jnp.int32),
            pltpu.VMEM((CH, D), jnp.float32),
            pltpu.VMEM((CH, D), jnp.float32),
            pltpu.VMEM((ZR, D), jnp.float32),
            pltpu.SemaphoreType.DMA,
            pltpu.SemaphoreType.DMA,
            pltpu.SemaphoreType.DMA,
            pltpu.SemaphoreType.DMA,
        ],
    )(table, gidx, sidx)


def _seg50k(table, gidx, sidx):
    return pl.kernel(
        _seg50k_body,
        out_type=jax.ShapeDtypeStruct((NC, NUP, D), jnp.float32),
        mesh=_MESH,
        scratch_types=[
            pltpu.VMEM_SHARED((ACCU_ROWS, D), jnp.float32),
            pltpu.VMEM((CH,), jnp.int32),
            pltpu.VMEM((CH,), jnp.int32),
            pltpu.VMEM((CH,), jnp.int32),
            pltpu.VMEM((CH,), jnp.int32),
            pltpu.VMEM((CH,), jnp.int32),
            pltpu.VMEM((CH, D), jnp.float32),
            pltpu.VMEM((CH, D), jnp.float32),
            pltpu.VMEM((ZR, D), jnp.float32),
            pltpu.SemaphoreType.DMA,
            pltpu.SemaphoreType.DMA,
            pltpu.SemaphoreType.DMA,
            pltpu.SemaphoreType.DMA,
        ],
    )(table, gidx, sidx)


def _lin_body(x_ref, w_ref, b_ref, o_ref):
    o_ref[...] = lax.dot_general(
        x_ref[...], w_ref[...], (((1,), (1,)), ((), ())),
        preferred_element_type=jnp.float32) + b_ref[...]


def _linear(x, w, b, bn=1000):
    n = x.shape[0]
    return pl.pallas_call(
        _lin_body,
        grid=(n // bn,),
        in_specs=[
            pl.BlockSpec((bn, D), lambda i: (i, 0)),
            pl.BlockSpec((D, D), lambda i: (0, 0)),
            pl.BlockSpec((1, D), lambda i: (0, 0)),
        ],
        out_specs=pl.BlockSpec((bn, D), lambda i: (i, 0)),
        out_shape=jax.ShapeDtypeStruct((n, D), jnp.float32),
    )(x, w, b.reshape(1, D))


def _fuse_body(p_ref, x_ref, wrel_ref, wroot_ref, b_ref, o_ref, *, relu, np_):
    acc = p_ref[0]
    for k in range(1, np_):
        acc = acc + p_ref[k]
    y = lax.dot_general(acc, wrel_ref[...], (((1,), (1,)), ((), ())),
                        preferred_element_type=jnp.float32)
    y = y + lax.dot_general(x_ref[...], wroot_ref[...], (((1,), (1,)), ((), ())),
                            preferred_element_type=jnp.float32)
    y = y + b_ref[...]
    if relu:
        y = jnp.maximum(y, 0.0)
    o_ref[...] = y


def _fuse(p, x, wrel, wroot, b, relu, bn=1000):
    n = x.shape[0]
    np_ = p.shape[0]
    return pl.pallas_call(
        functools.partial(_fuse_body, relu=relu, np_=np_),
        grid=(n // bn,),
        in_specs=[
            pl.BlockSpec((np_, bn, D), lambda i: (0, i, 0)),
            pl.BlockSpec((bn, D), lambda i: (i, 0)),
            pl.BlockSpec((D, D), lambda i: (0, 0)),
            pl.BlockSpec((D, D), lambda i: (0, 0)),
            pl.BlockSpec((1, D), lambda i: (0, 0)),
        ],
        out_specs=pl.BlockSpec((bn, D), lambda i: (i, 0)),
        out_shape=jax.ShapeDtypeStruct((n, D), jnp.float32),
    )(p, x, wrel, wroot, b.reshape(1, D))


def kernel(x_user, x_issue, edge_src, edge_dst, W_mlp, b_mlp,
           Wrel_ui1, brel_ui1, Wroot_ui1, Wrel_iu1, brel_iu1, Wroot_iu1,
           Wrel_ui2, brel_ui2, Wroot_ui2, Wrel_iu2, brel_iu2, Wroot_iu2):
    es = edge_src.astype(jnp.int32)
    ed = edge_dst.astype(jnp.int32)
    ar = jnp.arange(PAD, dtype=jnp.int32)
    es_g = jnp.concatenate([es, (ar * 7919) % NU])
    ed_g = jnp.concatenate([ed, (ar * 127) % NI])
    es_s = jnp.concatenate([es, jnp.full((PAD,), SENT, jnp.int32)])
    ed_s = jnp.concatenate([ed, NI + (ar % 8)])

    xi = _linear(x_issue, W_mlp, b_mlp)
    Pi1 = _seg10k(x_user, es_g, ed_s)
    h_i = _fuse(Pi1, xi, Wrel_ui1, Wroot_ui1, brel_ui1, relu=True)
    Pu1 = _seg50k(xi, ed_g, es_s)
    h_u = _fuse(Pu1, x_user, Wrel_iu1, Wroot_iu1, brel_iu1, relu=True)
    Pi2 = _seg10k(h_u, es_g, ed_s)
    o_i = _fuse(Pi2, h_i, Wrel_ui2, Wroot_ui2, brel_ui2, relu=False)
    Pu2 = _seg50k(h_i, ed_g, es_s)
    o_u = _fuse(Pu2, h_u, Wrel_iu2, Wroot_iu2, brel_iu2, relu=False)
    return o_i, o_u


# speedup vs baseline: 3.1171x; 1.1982x over previous
import functools

import jax
import jax.numpy as jnp
from jax import lax
from jax.experimental import pallas as pl
from jax.experimental.pallas import tpu as pltpu
from jax.experimental.pallas import tpu_sc as plsc

NU = 50000
NI = 10000
E = 320000
D = 128

NC, NS = 2, 16
NW = NC * NS
CH = 128
EC = E // CH
CPT = 2 * (-(-EC // (2 * NW)))
EPAD = CPT * NW * CH
PAD = EPAD - E

NIP = 10240
ACCI_ROWS = NIP

NR = 5
RNG = 10240
NUP = NR * RNG
SENT = NUP
TU = RNG // NS
ACCU_ROWS = RNG + 8

ZR = 64


def _zero_zbuf(zbuf):
    zv = jnp.zeros((16,), jnp.float32)

    def zrow(i, carry):
        for j in range(8):
            zbuf[i, pl.ds(j * 16, 16)] = zv
        return carry

    lax.fori_loop(0, ZR, zrow, 0)


def _seg10k_body(table, gidx, sidx, out, acc,
                 gba, gbb, sba, sbb, rowsa, rowsb, zbuf,
                 dsa, dsb, ssa, ssb, isa, isb):
    c = lax.axis_index("c")
    s = lax.axis_index("s")
    _zero_zbuf(zbuf)
    base = s * 640
    for k in range(10):
        pltpu.sync_copy(zbuf, acc.at[pl.ds(base + k * ZR, ZR)])
    plsc.subcore_barrier()

    chunk0 = (c * NS + s) * CPT
    rbufs = (rowsa, rowsb)
    dsems = (dsa, dsb)
    ssems = (ssa, ssb)
    gblks = (gba, gbb)
    sblks = (sba, sbb)
    isems = (isa, isb)

    def idx_copies(pair, b):
        eb = (chunk0 + pair * 2) * CH
        return (
            pltpu.make_async_copy(gidx.at[pl.ds(eb, 2 * CH)], gblks[b],
                                  isems[b]),
            pltpu.make_async_copy(sidx.at[pl.ds(eb, CH)], sblks[b].at[0],
                                  isems[b]),
            pltpu.make_async_copy(sidx.at[pl.ds(eb + CH, CH)], sblks[b].at[1],
                                  isems[b]),
        )

    def gref(k):
        b, slot = (k // 2) % 2, k % 2
        return gblks[b].at[pl.ds(slot * CH, CH)]

    pltpu.sync_copy(gidx.at[pl.ds(chunk0 * CH, 2 * CH)], gba)
    pltpu.sync_copy(sidx.at[pl.ds(chunk0 * CH, CH)], sba.at[0])
    pltpu.sync_copy(sidx.at[pl.ds((chunk0 + 1) * CH, CH)], sba.at[1])
    for cp in idx_copies(1, 1):
        cp.start()
    pltpu.async_copy(table.at[gref(0)], rowsa, dsa)

    def step4(i4, carry):
        c0 = i4 * 4
        for k in range(4):
            i = c0 + k
            jb = k % 2
            sref = sblks[(k // 2) % 2].at[k % 2]

            @pl.when((i >= 1) & (i + 1 < CPT))
            def _wait_prev_scatter(k=k):
                kp = (k + 1) % 2
                pltpu.make_async_copy(
                    rbufs[kp], acc.at[sblks[((k + 3) // 2) % 2].at[kp]],
                    ssems[kp]).wait()

            if k == 2:
                @pl.when(c0 + 4 < CPT)
                def _lda(i4=i4):
                    for cp in idx_copies(2 * i4 + 2, 0):
                        cp.start()

            @pl.when(i + 1 < CPT)
            def _next_gather(i4=i4, k=k):
                if k == 3:
                    for cp in idx_copies(2 * i4 + 2, 0):
                        cp.wait()
                if k == 1:
                    for cp in idx_copies(2 * i4 + 1, 1):
                        cp.wait()
                pltpu.async_copy(table.at[gref(k + 1)], rbufs[(k + 1) % 2],
                                 dsems[(k + 1) % 2])

            pltpu.make_async_copy(table.at[gref(k)], rbufs[jb],
                                  dsems[jb]).wait()
            pltpu.async_copy(rbufs[jb], acc.at[sref], ssems[jb], add=True)

            if k == 3:
                @pl.when(c0 + 6 < CPT)
                def _ldb(i4=i4):
                    for cp in idx_copies(2 * i4 + 3, 1):
                        cp.start()
        return carry

    lax.fori_loop(0, CPT // 4, step4, 0)
    pltpu.make_async_copy(rowsa, acc.at[sblks[1].at[0]], ssa).wait()
    pltpu.make_async_copy(rowsb, acc.at[sblks[1].at[1]], ssb).wait()
    plsc.subcore_barrier()
    pltpu.sync_copy(acc.at[pl.ds(base, 640)], out.at[c, pl.ds(base, 640)])


def _seg50k_body(table, gidx, sidx, out, acc,
                 gba, gbb, sba, sbb, sv2a, sv2b, rowsa, rowsb, zbuf,
                 dsa, dsb, ssa, ssb, isa, isb):
    c = lax.axis_index("c")
    s = lax.axis_index("s")
    _zero_zbuf(zbuf)
    trash = RNG + (lax.iota(jnp.int32, 16) & 7)
    chunk0 = (c * NS + s) * CPT
    base = s * TU
    rbufs = (rowsa, rowsb)
    dsems = (dsa, dsb)
    ssems = (ssa, ssb)
    gblks = (gba, gbb)
    sblks = (sba, sbb)
    sv2s = (sv2a, sv2b)
    isems = (isa, isb)

    def idx_copies(pair, b):
        eb = (chunk0 + pair * 2) * CH
        return (
            pltpu.make_async_copy(gidx.at[pl.ds(eb, 2 * CH)], gblks[b],
                                  isems[b]),
            pltpu.make_async_copy(sidx.at[pl.ds(eb, 2 * CH)], sblks[b],
                                  isems[b]),
        )

    def gref(k):
        b, slot = (k // 2) % 2, k % 2
        return gblks[b].at[pl.ds(slot * CH, CH)]

    for r in range(NR):
        for k in range(TU // ZR):
            pltpu.sync_copy(zbuf, acc.at[pl.ds(base + k * ZR, ZR)])
        plsc.subcore_barrier()

        pltpu.sync_copy(gidx.at[pl.ds(chunk0 * CH, 2 * CH)], gba)
        pltpu.sync_copy(sidx.at[pl.ds(chunk0 * CH, 2 * CH)], sba)
        for cp in idx_copies(1, 1):
            cp.start()
        pltpu.async_copy(table.at[gref(0)], rowsa, dsa)

        def step4(i4, carry, r=r):
            c0 = i4 * 4
            for k in range(4):
                i = c0 + k
                jb = k % 2

                @pl.when((i >= 1) & (i + 1 < CPT))
                def _wait_prev_scatter(k=k):
                    kp = (k + 1) % 2
                    pltpu.make_async_copy(rbufs[kp], acc.at[sv2s[kp]],
                                          ssems[kp]).wait()

                if k == 2:
                    @pl.when(c0 + 4 < CPT)
                    def _lda(i4=i4):
                        for cp in idx_copies(2 * i4 + 2, 0):
                            cp.start()

                @pl.when(i + 1 < CPT)
                def _next_gather(i4=i4, k=k):
                    if k == 3:
                        for cp in idx_copies(2 * i4 + 2, 0):
                            cp.wait()
                    if k == 1:
                        for cp in idx_copies(2 * i4 + 1, 1):
                            cp.wait()
                    pltpu.async_copy(table.at[gref(k + 1)],
                                     rbufs[(k + 1) % 2], dsems[(k + 1) % 2])

                sb = sblks[(k // 2) % 2]
                sv2 = sv2s[jb]
                for q in range(8):
                    v = sb[pl.ds((k % 2) * CH + q * 16, 16)]
                    loc = v - r * RNG
                    ok = (loc >= 0) & (loc < RNG)
                    sv2[pl.ds(q * 16, 16)] = jnp.where(ok, loc, trash)

                pltpu.make_async_copy(table.at[gref(k)], rbufs[jb],
                                      dsems[jb]).wait()
                pltpu.async_copy(rbufs[jb], acc.at[sv2], ssems[jb], add=True)

                if k == 3:
                    @pl.when(c0 + 6 < CPT)
                    def _ldb(i4=i4):
                        for cp in idx_copies(2 * i4 + 3, 1):
                            cp.start()
            return carry

        lax.fori_loop(0, CPT // 4, step4, 0)
        pltpu.make_async_copy(rowsa, acc.at[sv2a], ssa).wait()
        pltpu.make_async_copy(rowsb, acc.at[sv2b], ssb).wait()
        plsc.subcore_barrier()
        pltpu.sync_copy(acc.at[pl.ds(base, TU)],
                        out.at[c, pl.ds(r * RNG + base, TU)])
        plsc.subcore_barrier()


_MESH = plsc.VectorSubcoreMesh(core_axis_name="c", subcore_axis_name="s",
                               num_cores=NC, num_subcores=NS)


def _seg10k(table, gidx, sidx):
    return pl.kernel(
        _seg10k_body,
        out_type=jax.ShapeDtypeStruct((NC, NIP, D), jnp.float32),
        mesh=_MESH,
        scratch_types=[
            pltpu.VMEM_SHARED((ACCI_ROWS, D), jnp.float32),
            pltpu.VMEM((2 * CH,), jnp.int32),
            pltpu.VMEM((2 * CH,), jnp.int32),
            pltpu.VMEM((2, CH), jnp.int32),
            pltpu.VMEM((2, CH), jnp.int32),
            pltpu.VMEM((CH, D), jnp.float32),
            pltpu.VMEM((CH, D), jnp.float32),
            pltpu.VMEM((ZR, D), jnp.float32),
            pltpu.SemaphoreType.DMA,
            pltpu.SemaphoreType.DMA,
            pltpu.SemaphoreType.DMA,
            pltpu.SemaphoreType.DMA,
            pltpu.SemaphoreType.DMA,
            pltpu.SemaphoreType.DMA,
        ],
    )(table, gidx, sidx)


def _seg50k(table, gidx, sidx):
    return pl.kernel(
        _seg50k_body,
        out_type=jax.ShapeDtypeStruct((NC, NUP, D), jnp.float32),
        mesh=_MESH,
        scratch_types=[
            pltpu.VMEM_SHARED((ACCU_ROWS, D), jnp.float32),
            pltpu.VMEM((2 * CH,), jnp.int32),
            pltpu.VMEM((2 * CH,), jnp.int32),
            pltpu.VMEM((2 * CH,), jnp.int32),
            pltpu.VMEM((2 * CH,), jnp.int32),
            pltpu.VMEM((CH,), jnp.int32),
            pltpu.VMEM((CH,), jnp.int32),
            pltpu.VMEM((CH, D), jnp.float32),
            pltpu.VMEM((CH, D), jnp.float32),
            pltpu.VMEM((ZR, D), jnp.float32),
            pltpu.SemaphoreType.DMA,
            pltpu.SemaphoreType.DMA,
            pltpu.SemaphoreType.DMA,
            pltpu.SemaphoreType.DMA,
            pltpu.SemaphoreType.DMA,
            pltpu.SemaphoreType.DMA,
        ],
    )(table, gidx, sidx)


def _lin_body(x_ref, w_ref, b_ref, o_ref):
    o_ref[...] = lax.dot_general(
        x_ref[...], w_ref[...], (((1,), (1,)), ((), ())),
        preferred_element_type=jnp.float32) + b_ref[...]


def _linear(x, w, b, bn=1000):
    n = x.shape[0]
    return pl.pallas_call(
        _lin_body,
        grid=(n // bn,),
        in_specs=[
            pl.BlockSpec((bn, D), lambda i: (i, 0)),
            pl.BlockSpec((D, D), lambda i: (0, 0)),
            pl.BlockSpec((1, D), lambda i: (0, 0)),
        ],
        out_specs=pl.BlockSpec((bn, D), lambda i: (i, 0)),
        out_shape=jax.ShapeDtypeStruct((n, D), jnp.float32),
    )(x, w, b.reshape(1, D))


def _fuse_body(p_ref, x_ref, wrel_ref, wroot_ref, b_ref, o_ref, *, relu, np_):
    acc = p_ref[0]
    for k in range(1, np_):
        acc = acc + p_ref[k]
    y = lax.dot_general(acc, wrel_ref[...], (((1,), (1,)), ((), ())),
                        preferred_element_type=jnp.float32)
    y = y + lax.dot_general(x_ref[...], wroot_ref[...], (((1,), (1,)), ((), ())),
                            preferred_element_type=jnp.float32)
    y = y + b_ref[...]
    if relu:
        y = jnp.maximum(y, 0.0)
    o_ref[...] = y


def _fuse(p, x, wrel, wroot, b, relu, bn=1000):
    n = x.shape[0]
    np_ = p.shape[0]
    return pl.pallas_call(
        functools.partial(_fuse_body, relu=relu, np_=np_),
        grid=(n // bn,),
        in_specs=[
            pl.BlockSpec((np_, bn, D), lambda i: (0, i, 0)),
            pl.BlockSpec((bn, D), lambda i: (i, 0)),
            pl.BlockSpec((D, D), lambda i: (0, 0)),
            pl.BlockSpec((D, D), lambda i: (0, 0)),
            pl.BlockSpec((1, D), lambda i: (0, 0)),
        ],
        out_specs=pl.BlockSpec((bn, D), lambda i: (i, 0)),
        out_shape=jax.ShapeDtypeStruct((n, D), jnp.float32),
    )(p, x, wrel, wroot, b.reshape(1, D))


def kernel(x_user, x_issue, edge_src, edge_dst, W_mlp, b_mlp,
           Wrel_ui1, brel_ui1, Wroot_ui1, Wrel_iu1, brel_iu1, Wroot_iu1,
           Wrel_ui2, brel_ui2, Wroot_ui2, Wrel_iu2, brel_iu2, Wroot_iu2):
    es = edge_src.astype(jnp.int32)
    ed = edge_dst.astype(jnp.int32)
    ar = jnp.arange(PAD, dtype=jnp.int32)
    es_g = jnp.concatenate([es, (ar * 7919) % NU])
    ed_g = jnp.concatenate([ed, (ar * 127) % NI])
    es_s = jnp.concatenate([es, jnp.full((PAD,), SENT, jnp.int32)])
    ed_s = jnp.concatenate([ed, NI + (ar % 8)])

    xi = _linear(x_issue, W_mlp, b_mlp)
    Pi1 = _seg10k(x_user, es_g, ed_s)
    h_i = _fuse(Pi1, xi, Wrel_ui1, Wroot_ui1, brel_ui1, relu=True)
    Pu1 = _seg50k(xi, ed_g, es_s)
    h_u = _fuse(Pu1, x_user, Wrel_iu1, Wroot_iu1, brel_iu1, relu=True)
    Pi2 = _seg10k(h_u, es_g, ed_s)
    o_i = _fuse(Pi2, h_i, Wrel_ui2, Wroot_ui2, brel_ui2, relu=False)
    Pu2 = _seg50k(h_i, ed_g, es_s)
    o_u = _fuse(Pu2, h_u, Wrel_iu2, Wroot_iu2, brel_iu2, relu=False)
    return o_i, o_u
